# baseline (speedup 1.0000x reference)
import functools

import jax
import jax.numpy as jnp
from jax import lax
from jax.experimental import pallas as pl
from jax.experimental.pallas import tpu as pltpu
from jax.experimental.pallas import tpu_sc as plsc

NC = 2
NS = 16
NW = NC * NS


@functools.lru_cache(maxsize=None)
def _make_kernel(n_idx, gene_d, expr_d):
    gb = n_idx // NW
    gc = 16
    ec = 32
    nbuf = 4
    g_steps = gb // gc
    e_steps = gb // ec
    mesh = plsc.VectorSubcoreMesh(core_axis_name="c", subcore_axis_name="s")

    @functools.partial(
        pl.kernel,
        mesh=mesh,
        out_type=(
            jax.ShapeDtypeStruct((n_idx, gene_d), jnp.float32),
            jax.ShapeDtypeStruct((n_idx, expr_d), jnp.float32),
        ),
        scratch_types=(
            [pltpu.VMEM((gb,), jnp.int32)] * 2
            + [pltpu.VMEM((gc, gene_d), jnp.float32)] * nbuf
            + [pltpu.VMEM((ec, expr_d), jnp.float32)] * nbuf
            + [pltpu.SemaphoreType.DMA] * (2 * nbuf)
        ),
    )
    def emb_kernel(ids_hbm, bins_hbm, ptab_hbm, etab_hbm,
                   gene_out, expr_out, gidx_v, eidx_v, *scratch):
        gbufs = scratch[:nbuf]
        ebufs = scratch[nbuf:2 * nbuf]
        gsems = scratch[2 * nbuf:3 * nbuf]
        wsems = scratch[3 * nbuf:]
        wid = lax.axis_index("s") * NC + lax.axis_index("c")
        wbase = wid * gb
        pltpu.sync_copy(ids_hbm.at[pl.ds(wbase, gb)], gidx_v)
        pltpu.sync_copy(bins_hbm.at[pl.ds(wbase, gb)], eidx_v)

        def phase(idx_v, tab, out, bufs, chunk, steps):
            def body(j, carry):
                c0 = j * nbuf
                gs = [pltpu.async_copy(
                    tab.at[idx_v.at[pl.ds((c0 + b) * chunk, chunk)]],
                    bufs[b], gsems[b]) for b in range(nbuf)]
                ws = []
                for b in range(nbuf):
                    gs[b].wait()
                    ws.append(pltpu.async_copy(
                        bufs[b],
                        out.at[pl.ds(wbase + (c0 + b) * chunk, chunk)],
                        wsems[b]))
                for b in range(nbuf):
                    ws[b].wait()
                return carry

            lax.fori_loop(0, steps // nbuf, body, 0, unroll=False)

        phase(gidx_v, ptab_hbm, gene_out, gbufs, gc, g_steps)
        phase(eidx_v, etab_hbm, expr_out, ebufs, ec, e_steps)

    return emb_kernel


def kernel(input_ids, expr_bins, protein_emb, expr_table):
    b, l = input_ids.shape
    n = b * l
    ids = input_ids.reshape(n).astype(jnp.int32)
    bins = expr_bins.reshape(n).astype(jnp.int32)
    gene_d = protein_emb.shape[1]
    expr_d = expr_table.shape[1]
    emb = _make_kernel(n, gene_d, expr_d)
    gene, expr = emb(ids, bins, protein_emb, expr_table)
    return gene.reshape(b, l, gene_d), expr.reshape(b, l, expr_d)


# speedup vs baseline: 1.1101x; 1.1101x over previous
import functools

import jax
import jax.numpy as jnp
from jax import lax
from jax.experimental import pallas as pl
from jax.experimental.pallas import tpu as pltpu
from jax.experimental.pallas import tpu_sc as plsc

NC = 2
NS = 16
NW = NC * NS


@functools.lru_cache(maxsize=None)
def _make_gene_kernel(n_idx, gene_d):
    gb = n_idx // NW
    gc = 16
    nbuf = 4
    g_steps = gb // gc
    mesh = plsc.VectorSubcoreMesh(core_axis_name="c", subcore_axis_name="s")

    @functools.partial(
        pl.kernel,
        mesh=mesh,
        out_type=jax.ShapeDtypeStruct((n_idx, gene_d), jnp.float32),
        scratch_types=(
            [pltpu.VMEM((gb,), jnp.int32)]
            + [pltpu.VMEM((gc, gene_d), jnp.float32)] * nbuf
            + [pltpu.SemaphoreType.DMA] * (2 * nbuf)
        ),
    )
    def gene_kernel(ids_hbm, ptab_hbm, gene_out, gidx_v, *scratch):
        bufs = scratch[:nbuf]
        gsems = scratch[nbuf:2 * nbuf]
        wsems = scratch[2 * nbuf:]
        wid = lax.axis_index("s") * NC + lax.axis_index("c")
        wbase = wid * gb
        pltpu.sync_copy(ids_hbm.at[pl.ds(wbase, gb)], gidx_v)

        def body(j, carry):
            c0 = j * nbuf
            gs = [pltpu.async_copy(
                ptab_hbm.at[gidx_v.at[pl.ds((c0 + b) * gc, gc)]],
                bufs[b], gsems[b]) for b in range(nbuf)]
            ws = []
            for b in range(nbuf):
                gs[b].wait()
                ws.append(pltpu.async_copy(
                    bufs[b],
                    gene_out.at[pl.ds(wbase + (c0 + b) * gc, gc)],
                    wsems[b]))
            for b in range(nbuf):
                ws[b].wait()
            return carry

        lax.fori_loop(0, g_steps // nbuf, body, 0, unroll=False)

    return gene_kernel


@functools.lru_cache(maxsize=None)
def _make_expr_kernel(n_idx, vocab, expr_d):
    bn = 512
    nblk = n_idx // bn

    def body(idx_ref, tab_ref, out_ref):
        idx = idx_ref[0, 0, :]
        cols = lax.broadcasted_iota(jnp.int32, (bn, vocab), 1)
        onehot = (cols == idx[:, None]).astype(jnp.float32)
        out_ref[...] = lax.dot_general(
            onehot, tab_ref[...], (((1,), (0,)), ((), ())),
            preferred_element_type=jnp.float32)

    return pl.pallas_call(
        body,
        grid=(nblk,),
        in_specs=[
            pl.BlockSpec((1, 1, bn), lambda i: (i, 0, 0)),
            pl.BlockSpec((vocab, expr_d), lambda i: (0, 0)),
        ],
        out_specs=pl.BlockSpec((bn, expr_d), lambda i: (i, 0)),
        out_shape=jax.ShapeDtypeStruct((n_idx, expr_d), jnp.float32),
    )


def kernel(input_ids, expr_bins, protein_emb, expr_table):
    b, l = input_ids.shape
    n = b * l
    ids = input_ids.reshape(n).astype(jnp.int32)
    bins = expr_bins.reshape(n).astype(jnp.int32)
    gene_d = protein_emb.shape[1]
    vocab, expr_d = expr_table.shape
    gene = _make_gene_kernel(n, gene_d)(ids, protein_emb)
    bn = 512
    expr = _make_expr_kernel(n, vocab, expr_d)(
        bins.reshape(n // bn, 1, bn), expr_table)
    return gene.reshape(b, l, gene_d), expr.reshape(b, l, expr_d)


# speedup vs baseline: 1.1493x; 1.0353x over previous
import functools

import jax
import jax.numpy as jnp
from jax import lax
from jax.experimental import pallas as pl
from jax.experimental.pallas import tpu as pltpu
from jax.experimental.pallas import tpu_sc as plsc

NC = 2
NS = 16
NW = NC * NS


@functools.lru_cache(maxsize=None)
def _make_gene_kernel(n_idx, gene_d):
    gb = n_idx // NW
    gc = 16
    nbuf = 4
    g_steps = gb // gc
    mesh = plsc.VectorSubcoreMesh(core_axis_name="c", subcore_axis_name="s")

    @functools.partial(
        pl.kernel,
        mesh=mesh,
        out_type=jax.ShapeDtypeStruct((n_idx, gene_d), jnp.float32),
        scratch_types=(
            [pltpu.VMEM((gb,), jnp.int32)]
            + [pltpu.VMEM((gc, gene_d), jnp.float32)] * nbuf
            + [pltpu.SemaphoreType.DMA] * (2 * nbuf)
        ),
    )
    def gene_kernel(ids_hbm, ptab_hbm, gene_out, gidx_v, *scratch):
        bufs = scratch[:nbuf]
        gsems = scratch[nbuf:2 * nbuf]
        wsems = scratch[2 * nbuf:]
        wid = lax.axis_index("s") * NC + lax.axis_index("c")
        wbase = wid * gb
        pltpu.sync_copy(ids_hbm.at[pl.ds(wbase, gb)], gidx_v)

        def wait_write(b):
            pltpu.make_async_copy(
                bufs[b], gene_out.at[pl.ds(wbase, gc)], wsems[b]).wait()

        def body(j, carry):
            c0 = j * nbuf
            gs = []
            for b in range(nbuf):
                @pl.when(j > 0)
                def _(b=b):
                    wait_write(b)
                gs.append(pltpu.async_copy(
                    ptab_hbm.at[gidx_v.at[pl.ds((c0 + b) * gc, gc)]],
                    bufs[b], gsems[b]))
            for b in range(nbuf):
                gs[b].wait()
                pltpu.async_copy(
                    bufs[b],
                    gene_out.at[pl.ds(wbase + (c0 + b) * gc, gc)],
                    wsems[b])
            return carry

        lax.fori_loop(0, g_steps // nbuf, body, 0, unroll=False)
        for b in range(nbuf):
            wait_write(b)

    return gene_kernel


@functools.lru_cache(maxsize=None)
def _make_expr_kernel(n_idx, vocab, expr_d):
    bn = 512
    nblk = n_idx // bn

    def body(idx_ref, tab_ref, out_ref):
        idx = idx_ref[0, 0, :]
        cols = lax.broadcasted_iota(jnp.int32, (bn, vocab), 1)
        onehot = (cols == idx[:, None]).astype(jnp.float32)
        out_ref[...] = lax.dot_general(
            onehot, tab_ref[...], (((1,), (0,)), ((), ())),
            preferred_element_type=jnp.float32)

    return pl.pallas_call(
        body,
        grid=(nblk,),
        in_specs=[
            pl.BlockSpec((1, 1, bn), lambda i: (i, 0, 0)),
            pl.BlockSpec((vocab, expr_d), lambda i: (0, 0)),
        ],
        out_specs=pl.BlockSpec((bn, expr_d), lambda i: (i, 0)),
        out_shape=jax.ShapeDtypeStruct((n_idx, expr_d), jnp.float32),
    )


def kernel(input_ids, expr_bins, protein_emb, expr_table):
    b, l = input_ids.shape
    n = b * l
    ids = input_ids.reshape(n).astype(jnp.int32)
    bins = expr_bins.reshape(n).astype(jnp.int32)
    gene_d = protein_emb.shape[1]
    vocab, expr_d = expr_table.shape
    gene = _make_gene_kernel(n, gene_d)(ids, protein_emb)
    bn = 512
    expr = _make_expr_kernel(n, vocab, expr_d)(
        bins.reshape(n // bn, 1, bn), expr_table)
    return gene.reshape(b, l, gene_d), expr.reshape(b, l, expr_d)
